# Initial kernel scaffold; baseline (speedup 1.0000x reference)
import jax
import jax.numpy as jnp
from jax.experimental import pallas as pl


def kernel(x, W, positions, theta, signatures):
    raise NotImplementedError("write your pallas kernel here")


# speedup vs baseline: 1.9600x; 1.9600x over previous
import jax
import jax.numpy as jnp
import numpy as np
from jax.experimental import pallas as pl
from jax.experimental.pallas import tpu as pltpu

TOP_K = 8
N_BLK_Y = 256
N_BLK_S = 1024
K_BLK_S = 256


def _proj_kernel(w_hbm, x_ref, o_ref, w_vmem, sem):
    @pl.when(pl.program_id(0) == 0)
    def _():
        cp = pltpu.make_async_copy(w_hbm, w_vmem, sem)
        cp.start()
        cp.wait()
    o_ref[...] = jax.lax.dot_general(
        x_ref[...].astype(jnp.bfloat16), w_vmem[...],
        (((1,), (1,)), ((), ())), preferred_element_type=jnp.float32)


def _s_kernel(xp_ref, c_ref, o_ref):
    @pl.when(pl.program_id(1) == 0)
    def _init():
        o_ref[...] = jnp.zeros_like(o_ref)
    o_ref[...] += jax.lax.dot_general(
        xp_ref[...], c_ref[...], (((1,), (0,)), ((), ())),
        preferred_element_type=jnp.float32,
        precision=jax.lax.Precision.HIGHEST)


def _circulant(R, d):
    C = R[:, None]
    m = 1
    while m < d:
        C = jnp.concatenate([C, jnp.roll(C, m, axis=0)], axis=1)
        m *= 2
    return C


def kernel(x, W, positions, theta, signatures):
    N, d_model = x.shape
    d = theta.shape[0]

    E = signatures / jnp.maximum(
        jnp.linalg.norm(signatures, axis=-1, keepdims=True), 1e-12)
    phases = positions[:, None] * theta[None, :]
    L = jnp.fft.ifft(jnp.exp(1j * phases).astype(jnp.complex64), axis=-1).real
    fa = jnp.fft.fft(E.astype(jnp.complex64), axis=-1)
    fb = jnp.fft.fft(L.astype(jnp.complex64), axis=-1)
    R = jnp.fft.ifft(fa * fb, axis=-1).real.sum(axis=0)

    Wb = W.astype(jnp.bfloat16)
    y = pl.pallas_call(
        _proj_kernel,
        grid=(N // N_BLK_Y,),
        in_specs=[pl.BlockSpec(memory_space=pl.ANY),
                  pl.BlockSpec((N_BLK_Y, d_model), lambda i: (i, 0))],
        out_specs=pl.BlockSpec((N_BLK_Y, d), lambda i: (i, 0)),
        out_shape=jax.ShapeDtypeStruct((N, d), jnp.float32),
        scratch_shapes=[pltpu.VMEM((d, d_model), jnp.bfloat16),
                        pltpu.SemaphoreType.DMA],
    )(Wb, x)

    xp = y / jnp.maximum(jnp.linalg.norm(y, axis=-1, keepdims=True), 1e-12)

    C = _circulant(R, d)
    S = pl.pallas_call(
        _s_kernel,
        grid=(N // N_BLK_S, d // K_BLK_S),
        in_specs=[pl.BlockSpec((N_BLK_S, K_BLK_S), lambda i, k: (i, k)),
                  pl.BlockSpec((K_BLK_S, d), lambda i, k: (k, 0))],
        out_specs=pl.BlockSpec((N_BLK_S, d), lambda i, k: (i, 0)),
        out_shape=jax.ShapeDtypeStruct((N, d), jnp.float32),
    )(xp, C)

    Sn = S / jnp.maximum(jnp.linalg.norm(S, axis=-1, keepdims=True), 1e-12)

    scores = Sn @ E.T
    top_scores, indices = jax.lax.top_k(scores, TOP_K)
    weights = jax.nn.softmax(top_scores, axis=-1)
    return (weights, indices, scores)


# speedup vs baseline: 2.0700x; 1.0561x over previous
import jax
import jax.numpy as jnp
import numpy as np
from jax.experimental import pallas as pl
from jax.experimental.pallas import tpu as pltpu

TOP_K = 8
N_BLK_Y = 256
N_BLK_S = 1024
K_BLK_S = 256


def _proj_kernel(w_hbm, x_ref, o_ref, w_vmem, sem):
    @pl.when(pl.program_id(0) == 0)
    def _():
        cp = pltpu.make_async_copy(w_hbm, w_vmem, sem)
        cp.start()
        cp.wait()
    y = jax.lax.dot_general(
        x_ref[...].astype(jnp.bfloat16), w_vmem[...],
        (((1,), (1,)), ((), ())), preferred_element_type=jnp.float32)
    n = jnp.sqrt(jnp.sum(y * y, axis=1, keepdims=True))
    o_ref[...] = y / jnp.maximum(n, 1e-12)


def _s_kernel(xp_ref, c_ref, o_ref):
    @pl.when(pl.program_id(1) == 0)
    def _init():
        o_ref[...] = jnp.zeros_like(o_ref)
    o_ref[...] += jax.lax.dot_general(
        xp_ref[...], c_ref[...], (((1,), (0,)), ((), ())),
        preferred_element_type=jnp.float32,
        precision=jax.lax.Precision.HIGHEST)
    @pl.when(pl.program_id(1) == pl.num_programs(1) - 1)
    def _norm():
        s = o_ref[...]
        n = jnp.sqrt(jnp.sum(s * s, axis=1, keepdims=True))
        o_ref[...] = s / jnp.maximum(n, 1e-12)


def _circulant(R, d):
    C = R[:, None]
    m = 1
    while m < d:
        C = jnp.concatenate([C, jnp.roll(C, m, axis=0)], axis=1)
        m *= 2
    return C


def kernel(x, W, positions, theta, signatures):
    N, d_model = x.shape
    d = theta.shape[0]

    E = signatures / jnp.maximum(
        jnp.linalg.norm(signatures, axis=-1, keepdims=True), 1e-12)
    phases = positions[:, None] * theta[None, :]
    L = jnp.fft.ifft(jnp.exp(1j * phases).astype(jnp.complex64), axis=-1).real
    fa = jnp.fft.fft(E.astype(jnp.complex64), axis=-1)
    fb = jnp.fft.fft(L.astype(jnp.complex64), axis=-1)
    R = jnp.fft.ifft(fa * fb, axis=-1).real.sum(axis=0)

    Wb = W.astype(jnp.bfloat16)
    xp = pl.pallas_call(
        _proj_kernel,
        grid=(N // N_BLK_Y,),
        in_specs=[pl.BlockSpec(memory_space=pl.ANY),
                  pl.BlockSpec((N_BLK_Y, d_model), lambda i: (i, 0))],
        out_specs=pl.BlockSpec((N_BLK_Y, d), lambda i: (i, 0)),
        out_shape=jax.ShapeDtypeStruct((N, d), jnp.float32),
        scratch_shapes=[pltpu.VMEM((d, d_model), jnp.bfloat16),
                        pltpu.SemaphoreType.DMA],
    )(Wb, x)

    C = _circulant(R, d)
    Sn = pl.pallas_call(
        _s_kernel,
        grid=(N // N_BLK_S, d // K_BLK_S),
        in_specs=[pl.BlockSpec((N_BLK_S, K_BLK_S), lambda i, k: (i, k)),
                  pl.BlockSpec((K_BLK_S, d), lambda i, k: (k, 0))],
        out_specs=pl.BlockSpec((N_BLK_S, d), lambda i, k: (i, 0)),
        out_shape=jax.ShapeDtypeStruct((N, d), jnp.float32),
    )(xp, C)

    scores = Sn @ E.T
    top_scores, indices = jax.lax.top_k(scores, TOP_K)
    weights = jax.nn.softmax(top_scores, axis=-1)
    return (weights, indices, scores)


# speedup vs baseline: 2.0795x; 1.0046x over previous
import functools

import jax
import jax.numpy as jnp
import numpy as np
from jax import lax
from jax.experimental import pallas as pl
from jax.experimental.pallas import tpu as pltpu
from jax.experimental.pallas import tpu_sc as plsc

TOP_K = 8
LANES = 16
N_WORKERS = 32
N_BLK_Y = 256
N_BLK_S = 1024
K_BLK_S = 256


def _proj_kernel(w_hbm, x_ref, o_ref, w_vmem, sem):
    @pl.when(pl.program_id(0) == 0)
    def _():
        cp = pltpu.make_async_copy(w_hbm, w_vmem, sem)
        cp.start()
        cp.wait()
    y = jax.lax.dot_general(
        x_ref[...].astype(jnp.bfloat16), w_vmem[...],
        (((1,), (1,)), ((), ())), preferred_element_type=jnp.float32)
    n = jnp.sqrt(jnp.sum(y * y, axis=1, keepdims=True))
    o_ref[...] = y / jnp.maximum(n, 1e-12)


def _s_kernel(xp_ref, c_ref, o_ref):
    @pl.when(pl.program_id(1) == 0)
    def _init():
        o_ref[...] = jnp.zeros_like(o_ref)
    o_ref[...] += jax.lax.dot_general(
        xp_ref[...], c_ref[...], (((1,), (0,)), ((), ())),
        preferred_element_type=jnp.float32,
        precision=jax.lax.Precision.HIGHEST)
    @pl.when(pl.program_id(1) == pl.num_programs(1) - 1)
    def _norm():
        s = o_ref[...]
        n = jnp.sqrt(jnp.sum(s * s, axis=1, keepdims=True))
        o_ref[...] = s / jnp.maximum(n, 1e-12)


def _make_sc_topk(n_rows, n_experts):
    rows_w = n_rows // N_WORKERS
    groups = rows_w // LANES
    mesh = plsc.VectorSubcoreMesh(core_axis_name="c", subcore_axis_name="s")

    @functools.partial(
        pl.kernel, mesh=mesh,
        out_type=[jax.ShapeDtypeStruct((TOP_K, n_rows), jnp.float32),
                  jax.ShapeDtypeStruct((TOP_K, n_rows), jnp.int32)],
        scratch_types=[pltpu.VMEM((n_experts, rows_w), jnp.float32),
                       pltpu.VMEM((TOP_K, rows_w), jnp.float32),
                       pltpu.VMEM((TOP_K, rows_w), jnp.int32)],
    )
    def topk_kernel(scores_hbm, w_hbm, idx_hbm, sc_v, w_v, i_v):
        wid = lax.axis_index("s") * 2 + lax.axis_index("c")
        base = wid * rows_w
        pltpu.sync_copy(scores_hbm.at[:, pl.ds(base, rows_w)], sc_v)
        neg_inf = jnp.float32(-jnp.inf)

        def group_body(g, carry):
            col = g * LANES
            best_s = [jnp.full((LANES,), neg_inf, jnp.float32)
                      for _ in range(TOP_K)]
            best_i = [jnp.zeros((LANES,), jnp.int32) for _ in range(TOP_K)]
            for e in range(n_experts):
                cs = sc_v[e, pl.ds(col, LANES)]
                ci = jnp.full((LANES,), e, jnp.int32)
                for slot in range(TOP_K):
                    take = cs > best_s[slot]
                    ns = jnp.where(take, cs, best_s[slot])
                    ni = jnp.where(take, ci, best_i[slot])
                    cs = jnp.where(take, best_s[slot], cs)
                    ci = jnp.where(take, best_i[slot], ci)
                    best_s[slot] = ns
                    best_i[slot] = ni
            ex = [jnp.exp(s - best_s[0]) for s in best_s]
            tot = ex[0]
            for j in range(1, TOP_K):
                tot = tot + ex[j]
            for j in range(TOP_K):
                w_v[j, pl.ds(col, LANES)] = ex[j] / tot
                i_v[j, pl.ds(col, LANES)] = best_i[j]
            return carry

        lax.fori_loop(0, groups, group_body, 0)
        pltpu.sync_copy(w_v, w_hbm.at[:, pl.ds(base, rows_w)])
        pltpu.sync_copy(i_v, idx_hbm.at[:, pl.ds(base, rows_w)])

    return topk_kernel


def _circulant(R, d):
    C = R[:, None]
    m = 1
    while m < d:
        C = jnp.concatenate([C, jnp.roll(C, m, axis=0)], axis=1)
        m *= 2
    return C


def kernel(x, W, positions, theta, signatures):
    N, d_model = x.shape
    d = theta.shape[0]

    E = signatures / jnp.maximum(
        jnp.linalg.norm(signatures, axis=-1, keepdims=True), 1e-12)
    phases = positions[:, None] * theta[None, :]
    L = jnp.fft.ifft(jnp.exp(1j * phases).astype(jnp.complex64), axis=-1).real
    fa = jnp.fft.fft(E.astype(jnp.complex64), axis=-1)
    fb = jnp.fft.fft(L.astype(jnp.complex64), axis=-1)
    R = jnp.fft.ifft(fa * fb, axis=-1).real.sum(axis=0)

    Wb = W.astype(jnp.bfloat16)
    xp = pl.pallas_call(
        _proj_kernel,
        grid=(N // N_BLK_Y,),
        in_specs=[pl.BlockSpec(memory_space=pl.ANY),
                  pl.BlockSpec((N_BLK_Y, d_model), lambda i: (i, 0))],
        out_specs=pl.BlockSpec((N_BLK_Y, d), lambda i: (i, 0)),
        out_shape=jax.ShapeDtypeStruct((N, d), jnp.float32),
        scratch_shapes=[pltpu.VMEM((d, d_model), jnp.bfloat16),
                        pltpu.SemaphoreType.DMA],
    )(Wb, x)

    C = _circulant(R, d)
    Sn = pl.pallas_call(
        _s_kernel,
        grid=(N // N_BLK_S, d // K_BLK_S),
        in_specs=[pl.BlockSpec((N_BLK_S, K_BLK_S), lambda i, k: (i, k)),
                  pl.BlockSpec((K_BLK_S, d), lambda i, k: (k, 0))],
        out_specs=pl.BlockSpec((N_BLK_S, d), lambda i, k: (i, 0)),
        out_shape=jax.ShapeDtypeStruct((N, d), jnp.float32),
    )(xp, C)

    K = E.shape[0]
    scores = Sn @ E.T
    w_t, i_t = _make_sc_topk(N, K)(scores.T)
    weights = w_t.T
    indices = i_t.T
    return (weights, indices, scores)


# speedup vs baseline: 2.1050x; 1.0123x over previous
import functools

import jax
import jax.numpy as jnp
import numpy as np
from jax import lax
from jax.experimental import pallas as pl
from jax.experimental.pallas import tpu as pltpu
from jax.experimental.pallas import tpu_sc as plsc

TOP_K = 8
LANES = 16
N_WORKERS = 32
N_BLK_Y = 256
N_BLK_S = 1024
K_BLK_S = 256


def _proj_kernel(w_hbm, x_ref, o_ref, w_vmem, sem):
    @pl.when(pl.program_id(0) == 0)
    def _():
        cp = pltpu.make_async_copy(w_hbm, w_vmem, sem)
        cp.start()
        cp.wait()
    y = jax.lax.dot_general(
        x_ref[...].astype(jnp.bfloat16), w_vmem[...],
        (((1,), (1,)), ((), ())), preferred_element_type=jnp.float32)
    n = jnp.sqrt(jnp.sum(y * y, axis=1, keepdims=True))
    o_ref[...] = y / jnp.maximum(n, 1e-12)


def _s_kernel(xp_ref, c_ref, et_ref, o_ref, acc_ref):
    @pl.when(pl.program_id(1) == 0)
    def _init():
        acc_ref[...] = jnp.zeros_like(acc_ref)
    acc_ref[...] += jax.lax.dot_general(
        xp_ref[...], c_ref[...], (((1,), (0,)), ((), ())),
        preferred_element_type=jnp.float32,
        precision=jax.lax.Precision.HIGHEST)
    @pl.when(pl.program_id(1) == pl.num_programs(1) - 1)
    def _score():
        s = acc_ref[...]
        n = jnp.sqrt(jnp.sum(s * s, axis=1, keepdims=True))
        sn = s / jnp.maximum(n, 1e-12)
        o_ref[...] = jax.lax.dot_general(
            sn.astype(jnp.bfloat16), et_ref[...],
            (((1,), (0,)), ((), ())), preferred_element_type=jnp.float32)


def _make_sc_topk(n_rows, n_experts):
    rows_w = n_rows // N_WORKERS
    groups = rows_w // LANES
    mesh = plsc.VectorSubcoreMesh(core_axis_name="c", subcore_axis_name="s")

    @functools.partial(
        pl.kernel, mesh=mesh,
        out_type=[jax.ShapeDtypeStruct((TOP_K, n_rows), jnp.float32),
                  jax.ShapeDtypeStruct((TOP_K, n_rows), jnp.int32)],
        scratch_types=[pltpu.VMEM((n_experts, rows_w), jnp.float32),
                       pltpu.VMEM((TOP_K, rows_w), jnp.float32),
                       pltpu.VMEM((TOP_K, rows_w), jnp.int32)],
    )
    def topk_kernel(scores_hbm, w_hbm, idx_hbm, sc_v, w_v, i_v):
        wid = lax.axis_index("s") * 2 + lax.axis_index("c")
        base = wid * rows_w
        pltpu.sync_copy(scores_hbm.at[:, pl.ds(base, rows_w)], sc_v)
        neg_inf = jnp.float32(-jnp.inf)

        def group_body(g, carry):
            col = g * LANES
            best_s = [jnp.full((LANES,), neg_inf, jnp.float32)
                      for _ in range(TOP_K)]
            best_i = [jnp.zeros((LANES,), jnp.int32) for _ in range(TOP_K)]
            for e in range(n_experts):
                cs = sc_v[e, pl.ds(col, LANES)]
                ci = jnp.full((LANES,), e, jnp.int32)
                for slot in range(TOP_K):
                    take = cs > best_s[slot]
                    ns = jnp.where(take, cs, best_s[slot])
                    ni = jnp.where(take, ci, best_i[slot])
                    cs = jnp.where(take, best_s[slot], cs)
                    ci = jnp.where(take, best_i[slot], ci)
                    best_s[slot] = ns
                    best_i[slot] = ni
            ex = [jnp.exp(s - best_s[0]) for s in best_s]
            tot = ex[0]
            for j in range(1, TOP_K):
                tot = tot + ex[j]
            for j in range(TOP_K):
                w_v[j, pl.ds(col, LANES)] = ex[j] / tot
                i_v[j, pl.ds(col, LANES)] = best_i[j]
            return carry

        lax.fori_loop(0, groups, group_body, 0)
        pltpu.sync_copy(w_v, w_hbm.at[:, pl.ds(base, rows_w)])
        pltpu.sync_copy(i_v, idx_hbm.at[:, pl.ds(base, rows_w)])

    return topk_kernel


def _circulant(R, d):
    C = R[:, None]
    m = 1
    while m < d:
        C = jnp.concatenate([C, jnp.roll(C, m, axis=0)], axis=1)
        m *= 2
    return C


def kernel(x, W, positions, theta, signatures):
    N, d_model = x.shape
    d = theta.shape[0]

    E = signatures / jnp.maximum(
        jnp.linalg.norm(signatures, axis=-1, keepdims=True), 1e-12)
    phases = positions[:, None] * theta[None, :]
    L = jnp.fft.ifft(jnp.exp(1j * phases).astype(jnp.complex64), axis=-1).real
    fa = jnp.fft.fft(E.astype(jnp.complex64), axis=-1)
    fb = jnp.fft.fft(L.astype(jnp.complex64), axis=-1)
    R = jnp.fft.ifft(fa * fb, axis=-1).real.sum(axis=0)

    Wb = W.astype(jnp.bfloat16)
    xp = pl.pallas_call(
        _proj_kernel,
        grid=(N // N_BLK_Y,),
        in_specs=[pl.BlockSpec(memory_space=pl.ANY),
                  pl.BlockSpec((N_BLK_Y, d_model), lambda i: (i, 0))],
        out_specs=pl.BlockSpec((N_BLK_Y, d), lambda i: (i, 0)),
        out_shape=jax.ShapeDtypeStruct((N, d), jnp.float32),
        scratch_shapes=[pltpu.VMEM((d, d_model), jnp.bfloat16),
                        pltpu.SemaphoreType.DMA],
    )(Wb, x)

    K = E.shape[0]
    C = _circulant(R, d)
    Etb = jnp.zeros((d, 128), jnp.bfloat16).at[:, :K].set(
        E.astype(jnp.bfloat16).T)
    scores128 = pl.pallas_call(
        _s_kernel,
        grid=(N // N_BLK_S, d // K_BLK_S),
        in_specs=[pl.BlockSpec((N_BLK_S, K_BLK_S), lambda i, k: (i, k)),
                  pl.BlockSpec((K_BLK_S, d), lambda i, k: (k, 0)),
                  pl.BlockSpec((d, 128), lambda i, k: (0, 0))],
        out_specs=pl.BlockSpec((N_BLK_S, 128), lambda i, k: (i, 0)),
        out_shape=jax.ShapeDtypeStruct((N, 128), jnp.float32),
        scratch_shapes=[pltpu.VMEM((N_BLK_S, d), jnp.float32)],
    )(xp, C, Etb)
    scores = scores128[:, :K]

    w_t, i_t = _make_sc_topk(N, K)(scores.T)
    weights = w_t.T
    indices = i_t.T
    return (weights, indices, scores)
